# baseline (speedup 1.0000x reference)
import functools

import jax
import jax.numpy as jnp
from jax import lax
from jax.experimental import pallas as pl
from jax.experimental.pallas import tpu as pltpu
from jax.experimental.pallas import tpu_sc as plsc

VOCAB = 100000
D = 64
B = 4096
L = 50

NC, NS = 2, 16
NW = NC * NS
BPW = B // NW
LANES = 16
NBUF = 4
EPB = 4
RPB = EPB * L
NBLK = BPW // EPB


def _sc_lookup_mean(idx_blocks, table):
  mesh = plsc.VectorSubcoreMesh(core_axis_name="c", subcore_axis_name="s",
                                num_cores=NC, num_subcores=NS)

  @functools.partial(
      pl.kernel,
      out_type=jax.ShapeDtypeStruct((B, D), jnp.float32),
      mesh=mesh,
      compiler_params=pltpu.CompilerParams(use_tc_tiling_on_sc=False,
                                           needs_layout_passes=False),
      scratch_types=[
          pltpu.VMEM((NBLK, RPB), jnp.int32),
          pltpu.VMEM((NBUF, RPB, D // 2), jnp.int32),
          pltpu.VMEM((BPW, D), jnp.float32),
          [pltpu.SemaphoreType.DMA] * NBUF,
      ],
  )
  def k(idx_hbm, table_hbm, out_hbm, idx_v, rows_v, out_v, sems):
    wid = lax.axis_index("s") * NC + lax.axis_index("c")
    base = wid * BPW
    pltpu.sync_copy(idx_hbm.at[pl.ds(wid * NBLK, NBLK)], idx_v)

    lane = lax.iota(jnp.int32, LANES)
    even_lane = lax.rem(lane, jnp.int32(2)) == 0
    dup_lo = lax.div(lane, jnp.int32(2))
    dup_hi = dup_lo + jnp.int32(LANES // 2)

    def interleave(a, b, sel):
      return jnp.where(even_lane,
                       a.at[sel].get(mode="promise_in_bounds"),
                       b.at[sel].get(mode="promise_in_bounds"))

    def start(blk, j):
      pltpu.async_copy(
          table_hbm.at[idx_v.at[blk]], rows_v.at[j], sems[j])

    for j in range(NBUF):
      start(j, j)

    def body(i, _):
      for j in range(NBUF):
        blk = i * NBUF + j
        pltpu.make_async_copy(
            table_hbm.at[idx_v.at[0]], rows_v.at[j], sems[j]).wait()
        for p in range(EPB):
          e = blk * EPB + p
          for g in range(D // (2 * LANES)):
            acc_lo = jnp.zeros((LANES,), jnp.float32)
            acc_hi = jnp.zeros((LANES,), jnp.float32)
            for r in range(L):
              pair = rows_v[j, p * L + r, pl.ds(g * LANES, LANES)]
              acc_lo = acc_lo + plsc.bitcast(
                  lax.shift_left(pair, jnp.int32(16)), jnp.float32)
              acc_hi = acc_hi + plsc.bitcast(
                  lax.bitwise_and(pair, jnp.int32(-65536)), jnp.float32)
            acc_lo = acc_lo * jnp.float32(1.0 / L)
            acc_hi = acc_hi * jnp.float32(1.0 / L)
            out_v[e, pl.ds(g * 2 * LANES, LANES)] = interleave(
                acc_lo, acc_hi, dup_lo)
            out_v[e, pl.ds(g * 2 * LANES + LANES, LANES)] = interleave(
                acc_lo, acc_hi, dup_hi)
        start(jnp.minimum(blk + NBUF, NBLK - 1), j)
      return 0

    lax.fori_loop(0, NBLK // NBUF, body, 0)
    for j in range(NBUF):
      pltpu.make_async_copy(
          table_hbm.at[idx_v.at[0]], rows_v.at[j], sems[j]).wait()
    pltpu.sync_copy(out_v, out_hbm.at[pl.ds(base, BPW)])

  return k(idx_blocks, table)


def kernel(idx, embedding):
  idx_blocks = idx.astype(jnp.int32).reshape(B // EPB, RPB)
  packed = jax.lax.bitcast_convert_type(
      embedding.astype(jnp.bfloat16).reshape(VOCAB, D // 2, 2), jnp.int32)
  out = _sc_lookup_mean(idx_blocks, packed)
  return out[:, None, :]


# speedup vs baseline: 1.6788x; 1.6788x over previous
import functools

import jax
import jax.numpy as jnp
from jax import lax
from jax.experimental import pallas as pl
from jax.experimental.pallas import tpu as pltpu
from jax.experimental.pallas import tpu_sc as plsc

VOCAB = 100000
D = 64
B = 4096
L = 50

NC, NS = 2, 16
NW = NC * NS
BPW = B // NW
LANES = 16
NBUF = 4
EPB = 4
RPB = EPB * L
NBLK = BPW // EPB


def _sc_lookup_mean(idx_blocks, table):
  mesh = plsc.VectorSubcoreMesh(core_axis_name="c", subcore_axis_name="s",
                                num_cores=NC, num_subcores=NS)

  @functools.partial(
      pl.kernel,
      out_type=jax.ShapeDtypeStruct((B, D), jnp.float32),
      mesh=mesh,
      compiler_params=pltpu.CompilerParams(use_tc_tiling_on_sc=False,
                                           needs_layout_passes=False),
      scratch_types=[
          pltpu.VMEM((NBLK, RPB), jnp.int32),
          pltpu.VMEM((NBUF, RPB, D), jnp.float32),
          pltpu.VMEM((BPW, D), jnp.float32),
          [pltpu.SemaphoreType.DMA] * NBUF,
      ],
  )
  def k(idx_hbm, table_hbm, out_hbm, idx_v, rows_v, out_v, sems):
    wid = lax.axis_index("s") * NC + lax.axis_index("c")
    base = wid * BPW
    pltpu.sync_copy(idx_hbm.at[pl.ds(wid * NBLK, NBLK)], idx_v)

    def start(blk, j):
      pltpu.async_copy(
          table_hbm.at[idx_v.at[blk]], rows_v.at[j], sems[j])

    for j in range(NBUF):
      start(j, j)

    def body(i, _):
      for j in range(NBUF):
        blk = i * NBUF + j
        pltpu.make_async_copy(
            table_hbm.at[idx_v.at[0]], rows_v.at[j], sems[j]).wait()
        for p in range(EPB):
          e = blk * EPB + p
          for c in range(D // LANES):
            acc = jnp.zeros((LANES,), jnp.float32)
            for r in range(L):
              acc = acc + rows_v[j, p * L + r, pl.ds(c * LANES, LANES)]
            out_v[e, pl.ds(c * LANES, LANES)] = acc * jnp.float32(1.0 / L)
        start(jnp.minimum(blk + NBUF, NBLK - 1), j)
      return 0

    lax.fori_loop(0, NBLK // NBUF, body, 0)
    for j in range(NBUF):
      pltpu.make_async_copy(
          table_hbm.at[idx_v.at[0]], rows_v.at[j], sems[j]).wait()
    pltpu.sync_copy(out_v, out_hbm.at[pl.ds(base, BPW)])

  return k(idx_blocks, table)


def kernel(idx, embedding):
  idx_blocks = idx.astype(jnp.int32).reshape(B // EPB, RPB)
  out = _sc_lookup_mean(idx_blocks, embedding)
  return out[:, None, :]


# speedup vs baseline: 1.6835x; 1.0029x over previous
import functools

import jax
import jax.numpy as jnp
from jax import lax
from jax.experimental import pallas as pl
from jax.experimental.pallas import tpu as pltpu
from jax.experimental.pallas import tpu_sc as plsc

VOCAB = 100000
D = 64
B = 4096
L = 50

NC, NS = 2, 16
NW = NC * NS
BPW = B // NW
LANES = 16
NBUF = 4
EPB = 4
RPB = EPB * L
NBLK = BPW // EPB

VPW = VOCAB // NW
VCH = 125
NVCH = VPW // VCH

_MESH = plsc.VectorSubcoreMesh(core_axis_name="c", subcore_axis_name="s",
                               num_cores=NC, num_subcores=NS)
_PARAMS = pltpu.CompilerParams(use_tc_tiling_on_sc=False,
                               needs_layout_passes=False)


@functools.partial(
    pl.kernel,
    out_type=jax.ShapeDtypeStruct((VOCAB, D // 2), jnp.int32),
    mesh=_MESH,
    compiler_params=_PARAMS,
    scratch_types=[
        pltpu.VMEM((2, VCH, D), jnp.float32),
        pltpu.VMEM((2, VCH, D // 2), jnp.int32),
        [pltpu.SemaphoreType.DMA] * 2,
        [pltpu.SemaphoreType.DMA] * 2,
    ],
)
def _sc_pack(tab_hbm, out_hbm, in_v, out_v, in_sems, out_sems):
  wid = lax.axis_index("s") * NC + lax.axis_index("c")
  base = wid * VPW

  def start_in(ch, j):
    pltpu.async_copy(tab_hbm.at[pl.ds(base + ch * VCH, VCH)], in_v.at[j],
                     in_sems[j])

  def rne16(x):
    b = plsc.bitcast(x, jnp.int32)
    r = b + jnp.int32(0x7FFF) + lax.bitwise_and(
        lax.shift_right_logical(b, jnp.int32(16)), jnp.int32(1))
    return lax.shift_right_logical(r, jnp.int32(16))

  start_in(0, 0)
  start_in(1, 1)

  def body(ch, _):
    for j in range(2):
      c = ch * 2 + j
      pltpu.make_async_copy(tab_hbm.at[pl.ds(0, VCH)], in_v.at[j],
                            in_sems[j]).wait()
      if True:
        def row_body(r, _):
          for g in range(D // (2 * LANES)):
            lo = rne16(in_v[j, r, pl.ds(g * 2 * LANES, LANES)])
            hi = rne16(in_v[j, r, pl.ds(g * 2 * LANES + LANES, LANES)])
            out_v[j, r, pl.ds(g * LANES, LANES)] = lo + lax.shift_left(
                hi, jnp.int32(16))
          return 0

        lax.fori_loop(0, VCH, row_body, 0)
      pltpu.async_copy(out_v.at[j], out_hbm.at[pl.ds(base + c * VCH, VCH)],
                       out_sems[j])
      pltpu.make_async_copy(out_v.at[j],
                            out_hbm.at[pl.ds(0, VCH)], out_sems[j]).wait()
      start_in(jnp.minimum(c + 2, NVCH - 1), j)
    return 0

  lax.fori_loop(0, NVCH // 2, body, 0)
  for j in range(2):
    pltpu.make_async_copy(tab_hbm.at[pl.ds(0, VCH)], in_v.at[j],
                          in_sems[j]).wait()


def _sc_lookup_mean(idx_blocks, packed):
  @functools.partial(
      pl.kernel,
      out_type=jax.ShapeDtypeStruct((B, D), jnp.float32),
      mesh=_MESH,
      compiler_params=_PARAMS,
      scratch_types=[
          pltpu.VMEM((NBLK, RPB), jnp.int32),
          pltpu.VMEM((NBUF, RPB, D // 2), jnp.int32),
          pltpu.VMEM((BPW, D), jnp.float32),
          [pltpu.SemaphoreType.DMA] * NBUF,
      ],
  )
  def k(idx_hbm, table_hbm, out_hbm, idx_v, rows_v, out_v, sems):
    wid = lax.axis_index("s") * NC + lax.axis_index("c")
    base = wid * BPW
    pltpu.sync_copy(idx_hbm.at[pl.ds(wid * NBLK, NBLK)], idx_v)

    def start(blk, j):
      pltpu.async_copy(
          table_hbm.at[idx_v.at[blk]], rows_v.at[j], sems[j])

    for j in range(NBUF):
      start(j, j)

    def body(i, _):
      for j in range(NBUF):
        blk = i * NBUF + j
        pltpu.make_async_copy(
            table_hbm.at[idx_v.at[0]], rows_v.at[j], sems[j]).wait()
        for p in range(EPB):
          e = blk * EPB + p
          for g in range(D // (2 * LANES)):
            acc_lo = jnp.zeros((LANES,), jnp.float32)
            acc_hi = jnp.zeros((LANES,), jnp.float32)
            for r in range(L):
              pair = rows_v[j, p * L + r, pl.ds(g * LANES, LANES)]
              acc_lo = acc_lo + plsc.bitcast(
                  lax.shift_left(pair, jnp.int32(16)), jnp.float32)
              acc_hi = acc_hi + plsc.bitcast(
                  lax.bitwise_and(pair, jnp.int32(-65536)), jnp.float32)
            out_v[e, pl.ds(g * 2 * LANES, LANES)] = (
                acc_lo * jnp.float32(1.0 / L))
            out_v[e, pl.ds(g * 2 * LANES + LANES, LANES)] = (
                acc_hi * jnp.float32(1.0 / L))
        start(jnp.minimum(blk + NBUF, NBLK - 1), j)
      return 0

    lax.fori_loop(0, NBLK // NBUF, body, 0)
    for j in range(NBUF):
      pltpu.make_async_copy(
          table_hbm.at[idx_v.at[0]], rows_v.at[j], sems[j]).wait()
    pltpu.sync_copy(out_v, out_hbm.at[pl.ds(base, BPW)])

  return k(idx_blocks, packed)


def kernel(idx, embedding):
  idx_blocks = idx.astype(jnp.int32).reshape(B // EPB, RPB)
  out = _sc_lookup_mean(idx_blocks, _sc_pack(embedding))
  return out[:, None, :]


# speedup vs baseline: 2.0592x; 1.2231x over previous
import functools

import jax
import jax.numpy as jnp
from jax import lax
from jax.experimental import pallas as pl
from jax.experimental.pallas import tpu as pltpu
from jax.experimental.pallas import tpu_sc as plsc

VOCAB = 100000
D = 64
B = 4096
L = 50
LPAD = 56

NC, NS = 2, 16
NW = NC * NS
BPW = B // NW
LANES = 16
NBUF = 4
EPB = 2


def _sc_lookup_mean(idx_flat, table):
  mesh = plsc.VectorSubcoreMesh(core_axis_name="c", subcore_axis_name="s",
                                num_cores=NC, num_subcores=NS)

  @functools.partial(
      pl.kernel,
      out_type=jax.ShapeDtypeStruct((B, D), jnp.float32),
      mesh=mesh,
      compiler_params=pltpu.CompilerParams(use_tc_tiling_on_sc=False,
                                           needs_layout_passes=False),
      scratch_types=[
          pltpu.VMEM((BPW * LPAD,), jnp.int32),
          pltpu.VMEM((NBUF, EPB * LPAD, D), jnp.bfloat16),
          pltpu.VMEM((BPW, D), jnp.float32),
          [pltpu.SemaphoreType.DMA] * NBUF,
      ],
  )
  def k(idx_hbm, table_hbm, out_hbm, idx_v, rows_v, out_v, sems):
    wid = lax.axis_index("s") * NC + lax.axis_index("c")
    base = wid * BPW
    pltpu.sync_copy(idx_hbm.at[pl.ds(base * LPAD, BPW * LPAD)], idx_v)

    lane = lax.iota(jnp.int32, LANES)
    even_lane = lax.rem(lane, jnp.int32(2)) == 0
    dup_lo = lax.div(lane, jnp.int32(2))
    dup_hi = dup_lo + jnp.int32(LANES // 2)

    def interleave(a, b, sel):
      return jnp.where(even_lane,
                       a.at[sel].get(mode="promise_in_bounds"),
                       b.at[sel].get(mode="promise_in_bounds"))

    nblk = BPW // EPB

    def start(blk, j):
      pltpu.async_copy(
          table_hbm.at[idx_v.at[pl.ds(blk * EPB * LPAD, EPB * LPAD)]],
          rows_v.at[j], sems[j])

    for j in range(NBUF):
      start(j, j)

    def body(i, _):
      for j in range(NBUF):
        blk = i * NBUF + j
        pltpu.make_async_copy(
            table_hbm.at[idx_v.at[pl.ds(0, EPB * LPAD)]],
            rows_v.at[j], sems[j]).wait()
        for p in range(EPB):
          e = blk * EPB + p
          for g in range(D // (2 * LANES)):
            acc_lo = jnp.zeros((LANES,), jnp.float32)
            acc_hi = jnp.zeros((LANES,), jnp.float32)
            for r in range(L):
              chunk = rows_v[j, p * LPAD + r, pl.ds(g * 2 * LANES, 2 * LANES)]
              pair = plsc.bitcast(chunk, jnp.int32)
              acc_lo = acc_lo + plsc.bitcast(
                  lax.shift_left(pair, jnp.int32(16)), jnp.float32)
              acc_hi = acc_hi + plsc.bitcast(
                  lax.bitwise_and(pair, jnp.int32(-65536)), jnp.float32)
            acc_lo = acc_lo * jnp.float32(1.0 / L)
            acc_hi = acc_hi * jnp.float32(1.0 / L)
            out_v[e, pl.ds(g * 2 * LANES, LANES)] = interleave(
                acc_lo, acc_hi, dup_lo)
            out_v[e, pl.ds(g * 2 * LANES + LANES, LANES)] = interleave(
                acc_lo, acc_hi, dup_hi)
        start(jnp.minimum(blk + NBUF, nblk - 1), j)
      return 0

    lax.fori_loop(0, nblk // NBUF, body, 0)
    for j in range(NBUF):
      pltpu.make_async_copy(
          table_hbm.at[idx_v.at[pl.ds(0, EPB * LPAD)]],
          rows_v.at[j], sems[j]).wait()
    pltpu.sync_copy(out_v, out_hbm.at[pl.ds(base, BPW)])

  return k(idx_flat, table)


def kernel(idx, embedding):
  idx32 = idx.astype(jnp.int32)
  idx_pad = jnp.concatenate([idx32, idx32[:, : LPAD - L]], axis=1).reshape(-1)
  out = _sc_lookup_mean(idx_pad, embedding.astype(jnp.bfloat16))
  return out[:, None, :]
